# baseline (speedup 1.0000x reference)
import dataclasses

import jax
import jax.numpy as jnp
from jax import lax
from jax.experimental import pallas as pl
from jax.experimental.pallas import tpu as pltpu
from jax.experimental.pallas import tpu_sc as plsc

BATCH = 64
N = 8192
K = 8
L = 16
NC = 2
NS = 16
NW = NC * NS
ROWS_PER_W = BATCH // NW
NQ = 4
QELEMS = N // NQ
QCHUNKS = QELEMS // L
QCAP = QELEMS + L
NEG = float("-inf")


def _topk_body(x_hbm, vals_hbm, idx_hbm, xbuf, cand, vbuf, ibuf, *sems):
    wid = lax.axis_index("s") * NC + lax.axis_index("c")
    r0 = wid * ROWS_PER_W
    copies = [
        pltpu.async_copy(x_hbm.at[r0 + r], xbuf.at[pl.ds(r * N, N)], sems[r])
        for r in range(ROWS_PER_W)
    ]

    lane = lax.iota(jnp.int32, L)
    top8_mask = lane < K
    pinf = jnp.full((L,), jnp.inf, jnp.float32)
    neg = jnp.full((L,), NEG, jnp.float32)
    guard_abs = jnp.full((L,), ROWS_PER_W * N, jnp.int32)
    sixteen = jnp.full((L,), L, jnp.int32)
    qbase = [jnp.full((L,), q * QCAP, jnp.int32) for q in range(NQ)]

    for cp in copies:
        cp.wait()
    xbuf[pl.ds(ROWS_PER_W * N, L)] = neg

    def row_body(r, carry):
        base = r * N

        def amax_step(g, Ms):
            gb = base + g * (8 * L)
            return tuple(
                jnp.maximum(Ms[u], xbuf[pl.ds(gb + u * L, L)]) for u in range(8)
            )

        Ms = lax.fori_loop(0, N // (8 * L), amax_step, (neg,) * 8)
        ma = jnp.maximum(jnp.maximum(Ms[0], Ms[1]), jnp.maximum(Ms[2], Ms[3]))
        mb = jnp.maximum(jnp.maximum(Ms[4], Ms[5]), jnp.maximum(Ms[6], Ms[7]))
        sa, _ = plsc.sort_key_val(ma, lane, descending=True)
        sb, _ = plsc.sort_key_val(mb, lane, descending=False)
        top16 = jnp.maximum(sa, sb)
        s16, _ = plsc.sort_key_val(top16, lane, descending=True)
        t0 = jnp.min(jnp.where(top8_mask, s16, pinf))
        t0v = jnp.full((L,), t0)

        def bscan_step(c, carry):
            poss, ixs = carry
            new_pos = []
            new_ix = []
            for q in range(NQ):
                x = xbuf[pl.ds(base + q * QELEMS + c * L, L)]
                m = x >= t0v
                plsc.store_scatter(cand, [poss[q]], ixs[q], mask=m)
                new_pos.append(jnp.where(m, poss[q] + sixteen, poss[q]))
                new_ix.append(ixs[q] + sixteen)
            return tuple(new_pos), tuple(new_ix)

        pos0 = tuple(qbase[q] + lane for q in range(NQ))
        ix0 = tuple(jnp.full((L,), q * QELEMS, jnp.int32) + lane for q in range(NQ))
        poss, _ = plsc.parallel_loop(
            0, QCHUNKS, 1, unroll=2, carry=(pos0, ix0)
        )(bscan_step)

        rel = [poss[q] - qbase[q] for q in range(NQ)]
        mall = jnp.maximum(jnp.maximum(rel[0], rel[1]), jnp.maximum(rel[2], rel[3]))
        ntrips = jnp.max(mall) // L
        base_vec = jnp.full((L,), base, jnp.int32)

        def merge_step(j, carry):
            V, IV = carry
            slot_rel = j * L + lane
            for q in range(NQ):
                valid = slot_rel < rel[q]
                idxv = cand[pl.ds(q * QCAP + j * L, L)]
                gidx = jnp.where(valid, idxv + base_vec, guard_abs)
                vals = plsc.load_gather(xbuf, [gidx])
                idxv = jnp.where(valid, idxv, guard_abs)
                vs, ivs = plsc.sort_key_val(vals, idxv, descending=False)
                keep = V >= vs
                mv = jnp.where(keep, V, vs)
                mi = jnp.where(keep, IV, ivs)
                V, IV = plsc.sort_key_val(mv, mi, descending=True)
            return V, IV

        V, IV = lax.fori_loop(0, ntrips, merge_step, (neg, lane))

        vbuf[pl.ds(r * L, L)] = V
        ibuf[pl.ds(r * L, L)] = IV
        return carry

    lax.fori_loop(0, ROWS_PER_W, row_body, 0)

    out_copies = []
    for r in range(ROWS_PER_W):
        out_copies.append(
            pltpu.async_copy(
                vbuf.at[pl.ds(r * L, K)],
                vals_hbm.at[r0 + r],
                sems[ROWS_PER_W + 2 * r],
            )
        )
        out_copies.append(
            pltpu.async_copy(
                ibuf.at[pl.ds(r * L, K)],
                idx_hbm.at[r0 + r],
                sems[ROWS_PER_W + 2 * r + 1],
            )
        )
    for cp in out_copies:
        cp.wait()


def _compiler_params():
    cp = pltpu.CompilerParams()
    if "needs_layout_passes" in pltpu.CompilerParams.__dataclass_fields__:
        cp = dataclasses.replace(cp, needs_layout_passes=False)
    if "use_tc_tiling_on_sc" in pltpu.CompilerParams.__dataclass_fields__:
        cp = dataclasses.replace(cp, use_tc_tiling_on_sc=False)
    return cp


@jax.jit
def kernel(x):
    mesh = plsc.VectorSubcoreMesh(
        core_axis_name="c", subcore_axis_name="s", num_cores=NC, num_subcores=NS
    )
    vals, idx = pl.kernel(
        _topk_body,
        out_type=(
            jax.ShapeDtypeStruct((BATCH, K), jnp.float32),
            jax.ShapeDtypeStruct((BATCH, K), jnp.int32),
        ),
        mesh=mesh,
        scratch_types=[
            pltpu.VMEM((ROWS_PER_W * N + L,), jnp.float32),
            pltpu.VMEM((NQ * QCAP,), jnp.int32),
            pltpu.VMEM((ROWS_PER_W * L,), jnp.float32),
            pltpu.VMEM((ROWS_PER_W * L,), jnp.int32),
        ]
        + [pltpu.SemaphoreType.DMA] * (3 * ROWS_PER_W),
        compiler_params=_compiler_params(),
    )(x)
    return vals, idx.astype(jnp.int64)


# speedup vs baseline: 1.0164x; 1.0164x over previous
import dataclasses

import jax
import jax.numpy as jnp
from jax import lax
from jax.experimental import pallas as pl
from jax.experimental.pallas import tpu as pltpu
from jax.experimental.pallas import tpu_sc as plsc

BATCH = 64
N = 8192
K = 8
L = 16
NC = 2
NS = 16
NW = NC * NS
ROWS_PER_W = BATCH // NW
NQ = 4
QELEMS = N // NQ
QCHUNKS = QELEMS // L
QCAP = QELEMS + L
NEG = float("-inf")


def _topk_body(x_hbm, vals_hbm, idx_hbm, xbuf, cand, vbuf, ibuf, *sems):
    wid = lax.axis_index("s") * NC + lax.axis_index("c")
    r0 = wid * ROWS_PER_W
    H = N // 2
    copies = [
        pltpu.async_copy(
            x_hbm.at[r0, pl.ds(0, H)], xbuf.at[pl.ds(0, H)], sems[0]
        ),
        pltpu.async_copy(
            x_hbm.at[r0, pl.ds(H, H)], xbuf.at[pl.ds(H, H)], sems[1]
        ),
        pltpu.async_copy(
            x_hbm.at[r0 + 1], xbuf.at[pl.ds(N, N)], sems[2]
        ),
    ]

    lane = lax.iota(jnp.int32, L)
    top8_mask = lane < K
    pinf = jnp.full((L,), jnp.inf, jnp.float32)
    neg = jnp.full((L,), NEG, jnp.float32)
    guard_abs = jnp.full((L,), ROWS_PER_W * N, jnp.int32)
    sixteen = jnp.full((L,), L, jnp.int32)
    qbase = [jnp.full((L,), q * QCAP, jnp.int32) for q in range(NQ)]

    for cp in copies:
        cp.wait()
    xbuf[pl.ds(ROWS_PER_W * N, L)] = neg

    def row_body(r, carry):
        base = r * N

        def amax_step(g, Ms):
            gb = base + g * (16 * L)
            return tuple(
                jnp.maximum(Ms[u], xbuf[pl.ds(gb + 2 * u * L, L)])
                for u in range(8)
            )

        Ms = lax.fori_loop(0, N // (16 * L), amax_step, (neg,) * 8)
        ma = jnp.maximum(jnp.maximum(Ms[0], Ms[1]), jnp.maximum(Ms[2], Ms[3]))
        mb = jnp.maximum(jnp.maximum(Ms[4], Ms[5]), jnp.maximum(Ms[6], Ms[7]))
        sa, _ = plsc.sort_key_val(ma, lane, descending=True)
        sb, _ = plsc.sort_key_val(mb, lane, descending=False)
        top16 = jnp.maximum(sa, sb)
        s16, _ = plsc.sort_key_val(top16, lane, descending=True)
        t0 = jnp.min(jnp.where(top8_mask, s16, pinf))
        t0v = jnp.full((L,), t0)

        def bscan_step(c, carry):
            poss, ixs = carry
            new_pos = []
            new_ix = []
            for q in range(NQ):
                x = xbuf[pl.ds(base + q * QELEMS + c * L, L)]
                m = x >= t0v
                plsc.store_scatter(cand, [poss[q]], ixs[q], mask=m)
                new_pos.append(jnp.where(m, poss[q] + sixteen, poss[q]))
                new_ix.append(ixs[q] + sixteen)
            return tuple(new_pos), tuple(new_ix)

        pos0 = tuple(qbase[q] + lane for q in range(NQ))
        ix0 = tuple(jnp.full((L,), q * QELEMS, jnp.int32) + lane for q in range(NQ))
        poss, _ = plsc.parallel_loop(
            0, QCHUNKS, 1, unroll=2, carry=(pos0, ix0)
        )(bscan_step)

        rel = [poss[q] - qbase[q] for q in range(NQ)]
        mall = jnp.maximum(jnp.maximum(rel[0], rel[1]), jnp.maximum(rel[2], rel[3]))
        ntrips = jnp.max(mall) // L
        base_vec = jnp.full((L,), base, jnp.int32)

        def merge_step(j, carry):
            V, IV = carry
            slot_rel = j * L + lane
            for q in range(NQ):
                valid = slot_rel < rel[q]
                idxv = cand[pl.ds(q * QCAP + j * L, L)]
                gidx = jnp.where(valid, idxv + base_vec, guard_abs)
                vals = plsc.load_gather(xbuf, [gidx])
                idxv = jnp.where(valid, idxv, guard_abs)
                vs, ivs = plsc.sort_key_val(vals, idxv, descending=False)
                keep = V >= vs
                mv = jnp.where(keep, V, vs)
                mi = jnp.where(keep, IV, ivs)
                V, IV = plsc.sort_key_val(mv, mi, descending=True)
            return V, IV

        V, IV = lax.fori_loop(0, ntrips, merge_step, (neg, lane))

        vbuf[pl.ds(r * L, L)] = V
        ibuf[pl.ds(r * L, L)] = IV
        return carry

    lax.fori_loop(0, ROWS_PER_W, row_body, 0)

    out_copies = []
    for r in range(ROWS_PER_W):
        out_copies.append(
            pltpu.async_copy(
                vbuf.at[pl.ds(r * L, K)],
                vals_hbm.at[pl.ds((r0 + r) * K, K)],
                sems[3 + 2 * r],
            )
        )
        out_copies.append(
            pltpu.async_copy(
                ibuf.at[pl.ds(r * L, K)],
                idx_hbm.at[pl.ds((r0 + r) * K, K)],
                sems[3 + 2 * r + 1],
            )
        )
    for cp in out_copies:
        cp.wait()


def _compiler_params():
    cp = pltpu.CompilerParams()
    if "needs_layout_passes" in pltpu.CompilerParams.__dataclass_fields__:
        cp = dataclasses.replace(cp, needs_layout_passes=False)
    return cp


@jax.jit
def kernel(x):
    mesh = plsc.VectorSubcoreMesh(
        core_axis_name="c", subcore_axis_name="s", num_cores=NC, num_subcores=NS
    )
    vals, idx = pl.kernel(
        _topk_body,
        out_type=(
            jax.ShapeDtypeStruct((BATCH * K,), jnp.float32),
            jax.ShapeDtypeStruct((BATCH * K,), jnp.int32),
        ),
        mesh=mesh,
        scratch_types=[
            pltpu.VMEM((ROWS_PER_W * N + L,), jnp.float32),
            pltpu.VMEM((NQ * QCAP,), jnp.int32),
            pltpu.VMEM((ROWS_PER_W * L,), jnp.float32),
            pltpu.VMEM((ROWS_PER_W * L,), jnp.int32),
        ]
        + [pltpu.SemaphoreType.DMA] * (3 + 2 * ROWS_PER_W),
        compiler_params=_compiler_params(),
    )(x)
    return vals.reshape(BATCH, K), idx.reshape(BATCH, K).astype(jnp.int64)


# speedup vs baseline: 1.0186x; 1.0021x over previous
import dataclasses

import jax
import jax.numpy as jnp
from jax import lax
from jax.experimental import pallas as pl
from jax.experimental.pallas import tpu as pltpu
from jax.experimental.pallas import tpu_sc as plsc

BATCH = 64
N = 8192
K = 8
L = 16
NC = 2
NS = 16
NW = NC * NS
ROWS_PER_W = BATCH // NW
NQ = 4
QELEMS = N // NQ
QCHUNKS = QELEMS // L
QCAP = QELEMS + L
NEG = float("-inf")


def _topk_body(x_hbm, vals_hbm, idx_hbm, xbuf, cand, vbuf, ibuf, *sems):
    wid = lax.axis_index("s") * NC + lax.axis_index("c")
    r0 = wid * ROWS_PER_W
    H = N // 2
    copies = [
        pltpu.async_copy(
            x_hbm.at[r0, pl.ds(0, H)], xbuf.at[pl.ds(0, H)], sems[0]
        ),
        pltpu.async_copy(
            x_hbm.at[r0, pl.ds(H, H)], xbuf.at[pl.ds(H, H)], sems[1]
        ),
        pltpu.async_copy(
            x_hbm.at[r0 + 1], xbuf.at[pl.ds(N, N)], sems[2]
        ),
    ]

    lane = lax.iota(jnp.int32, L)
    top8_mask = lane < K
    pinf = jnp.full((L,), jnp.inf, jnp.float32)
    neg = jnp.full((L,), NEG, jnp.float32)
    guard_abs = jnp.full((L,), ROWS_PER_W * N, jnp.int32)
    sixteen = jnp.full((L,), L, jnp.int32)
    qbase = [jnp.full((L,), q * QCAP, jnp.int32) for q in range(NQ)]

    for cp in copies:
        cp.wait()
    xbuf[pl.ds(ROWS_PER_W * N, L)] = neg

    def row_body(r, carry):
        base = r * N

        def amax_step(g, Ms):
            gb = base + g * (16 * L)
            return tuple(
                jnp.maximum(Ms[u], xbuf[pl.ds(gb + 2 * u * L, L)])
                for u in range(8)
            )

        Ms = lax.fori_loop(0, N // (16 * L), amax_step, (neg,) * 8)
        ma = jnp.maximum(jnp.maximum(Ms[0], Ms[1]), jnp.maximum(Ms[2], Ms[3]))
        mb = jnp.maximum(jnp.maximum(Ms[4], Ms[5]), jnp.maximum(Ms[6], Ms[7]))
        sa, _ = plsc.sort_key_val(ma, lane, descending=True)
        sb, _ = plsc.sort_key_val(mb, lane, descending=False)
        top16 = jnp.maximum(sa, sb)
        s16, _ = plsc.sort_key_val(top16, lane, descending=True)
        t0 = jnp.min(jnp.where(top8_mask, s16, pinf))
        t0v = jnp.full((L,), t0)

        def bscan_step(c, carry):
            poss, cvec = carry
            new_pos = []
            for q in range(NQ):
                x = xbuf[pl.ds(base + q * QELEMS + c * L, L)]
                m = x >= t0v
                plsc.store_scatter(cand, [poss[q]], cvec, mask=m)
                new_pos.append(jnp.where(m, poss[q] + sixteen, poss[q]))
            return tuple(new_pos), cvec + sixteen

        pos0 = tuple(qbase[q] + lane for q in range(NQ))
        c0 = jnp.full((L,), 0, jnp.int32)
        poss, _ = plsc.parallel_loop(
            0, QCHUNKS, 1, unroll=2, carry=(pos0, c0)
        )(bscan_step)

        rel = [poss[q] - qbase[q] for q in range(NQ)]
        mall = jnp.maximum(jnp.maximum(rel[0], rel[1]), jnp.maximum(rel[2], rel[3]))
        ntrips = jnp.max(mall) // L
        base_vec = jnp.full((L,), base, jnp.int32)

        qoff = [jnp.full((L,), q * QELEMS, jnp.int32) + lane for q in range(NQ)]

        def merge_step(j, carry):
            V, IV = carry
            slot_rel = j * L + lane
            for q in range(NQ):
                valid = slot_rel < rel[q]
                raw = cand[pl.ds(q * QCAP + j * L, L)]
                idxv = jnp.where(valid, raw + qoff[q], guard_abs)
                gidx = jnp.where(valid, raw + qoff[q] + base_vec, guard_abs)
                vals = plsc.load_gather(xbuf, [gidx])
                vs, ivs = plsc.sort_key_val(vals, idxv, descending=False)
                keep = V >= vs
                mv = jnp.where(keep, V, vs)
                mi = jnp.where(keep, IV, ivs)
                V, IV = plsc.sort_key_val(mv, mi, descending=True)
            return V, IV

        V, IV = lax.fori_loop(0, ntrips, merge_step, (neg, lane))

        vbuf[pl.ds(r * L, L)] = V
        ibuf[pl.ds(r * L, L)] = IV
        return carry

    lax.fori_loop(0, ROWS_PER_W, row_body, 0)

    out_copies = []
    for r in range(ROWS_PER_W):
        out_copies.append(
            pltpu.async_copy(
                vbuf.at[pl.ds(r * L, K)],
                vals_hbm.at[pl.ds((r0 + r) * K, K)],
                sems[3 + 2 * r],
            )
        )
        out_copies.append(
            pltpu.async_copy(
                ibuf.at[pl.ds(r * L, K)],
                idx_hbm.at[pl.ds((r0 + r) * K, K)],
                sems[3 + 2 * r + 1],
            )
        )
    for cp in out_copies:
        cp.wait()


def _compiler_params():
    cp = pltpu.CompilerParams()
    if "needs_layout_passes" in pltpu.CompilerParams.__dataclass_fields__:
        cp = dataclasses.replace(cp, needs_layout_passes=False)
    return cp


@jax.jit
def kernel(x):
    mesh = plsc.VectorSubcoreMesh(
        core_axis_name="c", subcore_axis_name="s", num_cores=NC, num_subcores=NS
    )
    vals, idx = pl.kernel(
        _topk_body,
        out_type=(
            jax.ShapeDtypeStruct((BATCH * K,), jnp.float32),
            jax.ShapeDtypeStruct((BATCH * K,), jnp.int32),
        ),
        mesh=mesh,
        scratch_types=[
            pltpu.VMEM((ROWS_PER_W * N + L,), jnp.float32),
            pltpu.VMEM((NQ * QCAP,), jnp.int32),
            pltpu.VMEM((ROWS_PER_W * L,), jnp.float32),
            pltpu.VMEM((ROWS_PER_W * L,), jnp.int32),
        ]
        + [pltpu.SemaphoreType.DMA] * (3 + 2 * ROWS_PER_W),
        compiler_params=_compiler_params(),
    )(x)
    return vals.reshape(BATCH, K), idx.reshape(BATCH, K).astype(jnp.int64)


# speedup vs baseline: 1.0194x; 1.0008x over previous
import dataclasses

import jax
import jax.numpy as jnp
from jax import lax
from jax.experimental import pallas as pl
from jax.experimental.pallas import tpu as pltpu
from jax.experimental.pallas import tpu_sc as plsc

BATCH = 64
N = 8192
K = 8
L = 16
NC = 2
NS = 16
NW = NC * NS
ROWS_PER_W = BATCH // NW
NQ = 4
QELEMS = N // NQ
QCHUNKS = QELEMS // L
QCAP = QELEMS + L
NEG = float("-inf")


def _topk_body(x_hbm, vals_hbm, idx_hbm, xbuf, cand, vbuf, ibuf, *sems):
    wid = lax.axis_index("s") * NC + lax.axis_index("c")
    r0 = wid * ROWS_PER_W
    H = N // 2
    copies = [
        pltpu.async_copy(
            x_hbm.at[r0, pl.ds(0, H)], xbuf.at[pl.ds(0, H)], sems[0]
        ),
        pltpu.async_copy(
            x_hbm.at[r0, pl.ds(H, H)], xbuf.at[pl.ds(H, H)], sems[1]
        ),
        pltpu.async_copy(
            x_hbm.at[r0 + 1], xbuf.at[pl.ds(N, N)], sems[2]
        ),
    ]

    lane = lax.iota(jnp.int32, L)
    top8_mask = lane < K
    pinf = jnp.full((L,), jnp.inf, jnp.float32)
    neg = jnp.full((L,), NEG, jnp.float32)
    guard_abs = jnp.full((L,), ROWS_PER_W * N, jnp.int32)
    sixteen = jnp.full((L,), L, jnp.int32)
    qbase = [jnp.full((L,), q * QCAP, jnp.int32) for q in range(NQ)]

    for cp in copies:
        cp.wait()
    xbuf[pl.ds(ROWS_PER_W * N, L)] = neg

    def row_body(r, carry):
        base = r * N

        def amax_step(g, Ms):
            gb = base + g * (16 * L)
            return tuple(
                jnp.maximum(Ms[u], xbuf[pl.ds(gb + 2 * u * L, L)])
                for u in range(8)
            )

        Ms = lax.fori_loop(0, N // (16 * L), amax_step, (neg,) * 8)
        ma = jnp.maximum(jnp.maximum(Ms[0], Ms[1]), jnp.maximum(Ms[2], Ms[3]))
        mb = jnp.maximum(jnp.maximum(Ms[4], Ms[5]), jnp.maximum(Ms[6], Ms[7]))
        sa, _ = plsc.sort_key_val(ma, lane, descending=True)
        sb, _ = plsc.sort_key_val(mb, lane, descending=False)
        top16 = jnp.maximum(sa, sb)
        s16, _ = plsc.sort_key_val(top16, lane, descending=True)
        t0 = jnp.min(jnp.where(top8_mask, s16, pinf))
        t0v = jnp.full((L,), t0)

        def bscan_step(c, carry):
            poss, ixs = carry
            new_pos = []
            new_ix = []
            for q in range(NQ):
                x = xbuf[pl.ds(base + q * QELEMS + c * L, L)]
                m = x >= t0v
                plsc.store_scatter(cand, [poss[q]], ixs[q], mask=m)
                new_pos.append(jnp.where(m, poss[q] + sixteen, poss[q]))
                new_ix.append(ixs[q] + sixteen)
            return tuple(new_pos), tuple(new_ix)

        pos0 = tuple(qbase[q] + lane for q in range(NQ))
        ix0 = tuple(jnp.full((L,), q * QELEMS, jnp.int32) + lane for q in range(NQ))
        poss, _ = plsc.parallel_loop(
            0, QCHUNKS, 1, unroll=2, carry=(pos0, ix0)
        )(bscan_step)

        rel = [poss[q] - qbase[q] for q in range(NQ)]
        mall = jnp.maximum(jnp.maximum(rel[0], rel[1]), jnp.maximum(rel[2], rel[3]))
        ntrips = jnp.max(mall) // L
        base_vec = jnp.full((L,), base, jnp.int32)

        def merge_step(j, carry):
            V, IV = carry
            slot_rel = j * L + lane
            for q in range(NQ):
                valid = slot_rel < rel[q]
                idxv = cand[pl.ds(q * QCAP + j * L, L)]
                gidx = jnp.where(valid, idxv + base_vec, guard_abs)
                vals = plsc.load_gather(xbuf, [gidx])
                idxv = jnp.where(valid, idxv, guard_abs)
                vs, ivs = plsc.sort_key_val(vals, idxv, descending=False)
                keep = V >= vs
                mv = jnp.where(keep, V, vs)
                mi = jnp.where(keep, IV, ivs)
                V, IV = plsc.sort_key_val(mv, mi, descending=True)
            return V, IV

        V, IV = lax.fori_loop(0, ntrips, merge_step, (neg, lane))

        vbuf[pl.ds(r * L, L)] = V
        ibuf[pl.ds(r * L, L)] = IV
        return carry

    lax.fori_loop(0, ROWS_PER_W, row_body, 0)

    out_copies = []
    for r in range(ROWS_PER_W):
        out_copies.append(
            pltpu.async_copy(
                vbuf.at[pl.ds(r * L, K)],
                vals_hbm.at[pl.ds((r0 + r) * K, K)],
                sems[3 + 2 * r],
            )
        )
        out_copies.append(
            pltpu.async_copy(
                ibuf.at[pl.ds(r * L, K)],
                idx_hbm.at[pl.ds((r0 + r) * K, K)],
                sems[3 + 2 * r + 1],
            )
        )
    for cp in out_copies:
        cp.wait()


def _compiler_params():
    cp = pltpu.CompilerParams()
    if "needs_layout_passes" in pltpu.CompilerParams.__dataclass_fields__:
        cp = dataclasses.replace(cp, needs_layout_passes=False)
    return cp


@jax.jit
def kernel(x):
    mesh = plsc.VectorSubcoreMesh(
        core_axis_name="c", subcore_axis_name="s", num_cores=NC, num_subcores=NS
    )
    vals, idx = pl.kernel(
        _topk_body,
        out_type=(
            jax.ShapeDtypeStruct((BATCH * K,), jnp.float32),
            jax.ShapeDtypeStruct((BATCH * K,), jnp.int32),
        ),
        mesh=mesh,
        scratch_types=[
            pltpu.VMEM((ROWS_PER_W * N + L,), jnp.float32),
            pltpu.VMEM((NQ * QCAP,), jnp.int32),
            pltpu.VMEM((ROWS_PER_W * L,), jnp.float32),
            pltpu.VMEM((ROWS_PER_W * L,), jnp.int32),
        ]
        + [pltpu.SemaphoreType.DMA] * (3 + 2 * ROWS_PER_W),
        compiler_params=_compiler_params(),
    )(x)
    return vals.reshape(BATCH, K), idx.reshape(BATCH, K).astype(jnp.int64)
